# baseline (speedup 1.0000x reference)
import jax
import jax.numpy as jnp
from jax.experimental import pallas as pl
from jax.experimental.pallas import tpu as pltpu

E = 64
DIM = 1024
HID = 1024
TOKENS = 8192
TILE = 136
ALPHA = 1.702
LIMIT = 7.0


def _half_ffn(xt, w1_half, b1_half, q_ref):
    dn = (((1,), (1,)), ((), ()))
    h = jax.lax.dot_general(xt, w1_half, dn,
                            preferred_element_type=jnp.float32)
    h = h + b1_half
    g = jnp.minimum(h, LIMIT)
    g = g * jax.nn.sigmoid(ALPHA * g)
    l = jnp.clip(h, -LIMIT, LIMIT) + 1.0
    act_full = g * jnp.roll(l, -1, axis=1)
    return jax.lax.dot_general(act_full.astype(jnp.bfloat16), q_ref[...],
                               (((1,), (0,)), ((), ())),
                               preferred_element_type=jnp.float32)


def _moe_kernel(counts_ref, x_ref, w1a_ref, w1b_ref, b1a_ref, b1b_ref,
                w2a_ref, w2b_ref, b2_ref, out_ref, q_ref):
    e = pl.program_id(0)

    @pl.when(e == 0)
    def _init():
        out_ref[...] = jnp.zeros_like(out_ref)
        k = jax.lax.broadcasted_iota(jnp.int32, (HID, HID // 2), 0)
        j = jax.lax.broadcasted_iota(jnp.int32, (HID, HID // 2), 1)
        q_ref[...] = (k == 2 * j).astype(jnp.bfloat16)

    start = jax.lax.fori_loop(
        0, e, lambda i, s: s + counts_ref[i], jnp.int32(0))
    count = counts_ref[e]
    base = (start // 8) * 8
    lo = start - base

    xt = x_ref[...]
    dn = (((1,), (1,)), ((), ()))

    act_a = _half_ffn(xt, w1a_ref[0, 0], b1a_ref[0, 0], q_ref)
    act_b = _half_ffn(xt, w1b_ref[0, 0], b1b_ref[0, 0], q_ref)

    out = jax.lax.dot_general(act_a, w2a_ref[0], dn,
                              preferred_element_type=jnp.float32)
    out = out + jax.lax.dot_general(act_b, w2b_ref[0], dn,
                                    preferred_element_type=jnp.float32)
    out = out + b2_ref[0]

    row = jax.lax.broadcasted_iota(jnp.int32, (TILE, 1), 0)
    out = jnp.where((row >= lo) & (row < lo + count), out, 0.0)
    out_ref[pl.ds(base, TILE), :] += out


@jax.jit
def kernel(x, mlp1_weight, mlp1_bias, mlp2_weight, mlp2_bias,
           num_tokens_per_expert):
    counts = num_tokens_per_expert.astype(jnp.int32)
    w1_4d = mlp1_weight.reshape(E, 2, HID, DIM)
    b1_4d = mlp1_bias.reshape(E, 2, 1, HID)

    def x_index(e, c):
        start = jax.lax.fori_loop(0, e, lambda i, s: s + c[i], jnp.int32(0))
        return (start // 8) * 8, 0

    grid_spec = pltpu.PrefetchScalarGridSpec(
        num_scalar_prefetch=1,
        grid=(E,),
        in_specs=[
            pl.BlockSpec((pl.Element(TILE), pl.Element(DIM)), x_index),
            pl.BlockSpec((1, 1, HID, DIM), lambda e, c: (e, 0, 0, 0)),
            pl.BlockSpec((1, 1, HID, DIM), lambda e, c: (e, 1, 0, 0)),
            pl.BlockSpec((1, 1, 1, HID), lambda e, c: (e, 0, 0, 0)),
            pl.BlockSpec((1, 1, 1, HID), lambda e, c: (e, 1, 0, 0)),
            pl.BlockSpec((1, DIM, HID // 2), lambda e, c: (e, 0, 0)),
            pl.BlockSpec((1, DIM, HID // 2), lambda e, c: (e, 0, 1)),
            pl.BlockSpec((1, 1, DIM), lambda e, c: (e, 0, 0)),
        ],
        out_specs=pl.BlockSpec((TOKENS, DIM), lambda e, c: (0, 0)),
        scratch_shapes=[pltpu.VMEM((HID, HID // 2), jnp.bfloat16)],
    )

    return pl.pallas_call(
        _moe_kernel,
        grid_spec=grid_spec,
        out_shape=jax.ShapeDtypeStruct((TOKENS, DIM), x.dtype),
        compiler_params=pltpu.CompilerParams(
            vmem_limit_bytes=120 * 1024 * 1024,
        ),
    )(counts, x, w1_4d, w1_4d, b1_4d, b1_4d, mlp2_weight, mlp2_weight,
      mlp2_bias.reshape(E, 1, DIM))


# speedup vs baseline: 1.0185x; 1.0185x over previous
import jax
import jax.numpy as jnp
from jax.experimental import pallas as pl
from jax.experimental.pallas import tpu as pltpu

E = 64
DIM = 1024
HID = 1024
TOKENS = 8192
TILE = 136
ALPHA = 1.702
LIMIT = 7.0


def _moe_kernel(counts_ref, x_ref, w1_ref, b1_ref, w2_ref, b2_ref, out_ref,
                q_ref):
    e = pl.program_id(0)

    @pl.when(e == 0)
    def _init():
        out_ref[...] = jnp.zeros_like(out_ref)
        k = jax.lax.broadcasted_iota(jnp.int32, (2 * HID, HID), 0)
        j = jax.lax.broadcasted_iota(jnp.int32, (2 * HID, HID), 1)
        q_ref[...] = (k == 2 * j).astype(jnp.bfloat16)

    start = jax.lax.fori_loop(
        0, e, lambda i, s: s + counts_ref[i], jnp.int32(0))
    count = counts_ref[e]
    base = (start // 8) * 8
    lo = start - base

    xt = x_ref[...]

    dn = (((1,), (1,)), ((), ()))
    h = jax.lax.dot_general(xt, w1_ref[0], dn,
                            preferred_element_type=jnp.float32)
    h = h + b1_ref[0]

    g = jnp.minimum(h, LIMIT)
    g = g * jax.nn.sigmoid(ALPHA * g)
    l = jnp.clip(h, -LIMIT, LIMIT) + 1.0
    act_full = g * jnp.roll(l, -1, axis=1)

    act_c = jax.lax.dot_general(act_full.astype(jnp.bfloat16), q_ref[...],
                                (((1,), (0,)), ((), ())),
                                preferred_element_type=jnp.float32)

    out = jax.lax.dot_general(act_c, w2_ref[0], dn,
                              preferred_element_type=jnp.float32)
    out = out + b2_ref[0]

    row = jax.lax.broadcasted_iota(jnp.int32, (TILE, 1), 0)
    out = jnp.where((row >= lo) & (row < lo + count), out, 0.0)
    out_ref[pl.ds(base, TILE), :] += out


@jax.jit
def kernel(x, mlp1_weight, mlp1_bias, mlp2_weight, mlp2_bias,
           num_tokens_per_expert):
    counts = num_tokens_per_expert.astype(jnp.int32)

    def x_index(e, c):
        start = jax.lax.fori_loop(0, e, lambda i, s: s + c[i], jnp.int32(0))
        return (start // 8) * 8, 0

    grid_spec = pltpu.PrefetchScalarGridSpec(
        num_scalar_prefetch=1,
        grid=(E,),
        in_specs=[
            pl.BlockSpec((pl.Element(TILE), pl.Element(DIM)), x_index),
            pl.BlockSpec((1, 2 * HID, DIM), lambda e, c: (e, 0, 0)),
            pl.BlockSpec((1, 1, 2 * HID), lambda e, c: (e, 0, 0)),
            pl.BlockSpec((1, DIM, HID), lambda e, c: (e, 0, 0)),
            pl.BlockSpec((1, 1, DIM), lambda e, c: (e, 0, 0)),
        ],
        out_specs=pl.BlockSpec((TOKENS, DIM), lambda e, c: (0, 0)),
        scratch_shapes=[pltpu.VMEM((2 * HID, HID), jnp.bfloat16)],
    )

    return pl.pallas_call(
        _moe_kernel,
        grid_spec=grid_spec,
        out_shape=jax.ShapeDtypeStruct((TOKENS, DIM), x.dtype),
        compiler_params=pltpu.CompilerParams(
            vmem_limit_bytes=120 * 1024 * 1024,
        ),
    )(counts, x, mlp1_weight, mlp1_bias.reshape(E, 1, 2 * HID), mlp2_weight,
      mlp2_bias.reshape(E, 1, DIM))
